# baseline (speedup 1.0000x reference)
import jax
import jax.numpy as jnp
from jax.experimental import pallas as pl
from jax.experimental.pallas import tpu as pltpu

N = 8192
BM1 = 512
NI1 = N // BM1
BM2 = 256
TAIL = 2048
NH = N - TAIL
NIH = NH // BM2


def _layer1_kernel(adj_ref, x_ref, w1_ref, b1_ref, w2_ref, abf_ref, u2_ref,
                   u1_s):
    @pl.when(pl.program_id(0) == 0)
    def _proj():
        u1_s[...] = jnp.dot(x_ref[...], w1_ref[...],
                            preferred_element_type=jnp.float32
                            ).astype(jnp.bfloat16)
    a = adj_ref[...].astype(jnp.bfloat16)
    abf_ref[...] = a
    h = jnp.maximum(
        jnp.dot(a, u1_s[...], preferred_element_type=jnp.float32)
        + b1_ref[...], 0.0)
    u2_ref[...] = jnp.dot(h, w2_ref[...], preferred_element_type=jnp.float32
                          ).astype(jnp.bfloat16)


def _layer23_kernel(abf_ref, atail_ref, u2_ref, b2_ref, w3_ref, b3_ref,
                    wr_ref, br_ref, o_ref, s_ref):
    l = pl.program_id(0)
    i = pl.program_id(1)
    row = pl.ds(i * BM2, BM2)

    @pl.when(l == 0)
    def _layer2():
        h = jnp.maximum(
            jnp.dot(abf_ref[...], u2_ref[...],
                    preferred_element_type=jnp.float32)
            + b2_ref[...], 0.0)
        s_ref[row, 0:1] = jnp.dot(h, w3_ref[...],
                                  preferred_element_type=jnp.float32)

    @pl.when(jnp.logical_and(l == 0, i == NIH - 1))
    def _layer2_tail():
        for c in range(TAIL // 512):
            h = jnp.maximum(
                jnp.dot(atail_ref[pl.ds(c * 512, 512), :], u2_ref[...],
                        preferred_element_type=jnp.float32)
                + b2_ref[...], 0.0)
            s_ref[pl.ds(NH + c * 512, 512), 0:1] = jnp.dot(
                h, w3_ref[...], preferred_element_type=jnp.float32)

    @pl.when(l == 1)
    def _layer3():
        v = s_ref[:, 0:1].astype(jnp.bfloat16)
        h = jnp.maximum(
            jnp.dot(abf_ref[...], v, preferred_element_type=jnp.float32)
            + b3_ref[...], 0.0)
        s_ref[row, 1:2] = h

    @pl.when(jnp.logical_and(l == 1, i == NIH - 1))
    def _layer3_tail_and_readout():
        v = s_ref[:, 0:1].astype(jnp.bfloat16)
        for c in range(TAIL // 512):
            h = jnp.maximum(
                jnp.dot(atail_ref[pl.ds(c * 512, 512), :], v,
                        preferred_element_type=jnp.float32)
                + b3_ref[...], 0.0)
            s_ref[pl.ds(NH + c * 512, 512), 1:2] = h
        o_ref[...] = br_ref[...] + jnp.sum(
            wr_ref[...] * s_ref[:, 1:2].reshape(1, N), keepdims=True)


def kernel(x, adj, W1, b1, W2, b2, W3, b3, Wr, br):
    adj_bf, u2 = pl.pallas_call(
        _layer1_kernel,
        grid=(NI1,),
        in_specs=[
            pl.BlockSpec((BM1, N), lambda i: (i, 0)),
            pl.BlockSpec((N, 128), lambda i: (0, 0)),
            pl.BlockSpec((128, 32), lambda i: (0, 0)),
            pl.BlockSpec((1, 32), lambda i: (0, 0)),
            pl.BlockSpec((32, 16), lambda i: (0, 0)),
        ],
        scratch_shapes=[pltpu.VMEM((N, 32), jnp.bfloat16)],
        out_specs=[
            pl.BlockSpec((BM1, N), lambda i: (i, 0)),
            pl.BlockSpec((BM1, 16), lambda i: (i, 0)),
        ],
        out_shape=[
            jax.ShapeDtypeStruct((N, N), jnp.bfloat16),
            jax.ShapeDtypeStruct((N, 16), jnp.bfloat16),
        ],
    )(adj, x, W1, b1.reshape(1, 32), W2)
    out = pl.pallas_call(
        _layer23_kernel,
        grid=(2, NIH),
        in_specs=[
            pl.BlockSpec((BM2, N), lambda l, i: (i, 0)),
            pl.BlockSpec((TAIL, N), lambda l, i: (NH // TAIL, 0)),
            pl.BlockSpec((N, 16), lambda l, i: (0, 0)),
            pl.BlockSpec((1, 16), lambda l, i: (0, 0)),
            pl.BlockSpec((16, 1), lambda l, i: (0, 0)),
            pl.BlockSpec((1, 1), lambda l, i: (0, 0)),
            pl.BlockSpec((1, N), lambda l, i: (0, 0)),
            pl.BlockSpec((1, 1), lambda l, i: (0, 0)),
        ],
        out_specs=pl.BlockSpec((1, 1), lambda l, i: (0, 0)),
        out_shape=jax.ShapeDtypeStruct((1, 1), jnp.float32),
        scratch_shapes=[
            pltpu.VMEM((N, 128), jnp.float32),
        ],
    )(adj_bf, adj_bf, u2, b2.reshape(1, 16), W3, b3.reshape(1, 1), Wr,
      br.reshape(1, 1))
    return out.reshape(1)


# speedup vs baseline: 1.1180x; 1.1180x over previous
import jax
import jax.numpy as jnp
from jax.experimental import pallas as pl
from jax.experimental.pallas import tpu as pltpu

N = 8192
BM1 = 512
NI1 = N // BM1
BM2 = 1024
TAIL = 1024
NH = N - TAIL
NIH = NH // BM2


def _layer1_kernel(adj_ref, x_ref, w1_ref, b1_ref, w2_ref, abf_ref, u2_ref,
                   u1_s):
    @pl.when(pl.program_id(0) == 0)
    def _proj():
        u1_s[...] = jnp.dot(x_ref[...], w1_ref[...],
                            preferred_element_type=jnp.float32
                            ).astype(jnp.bfloat16)
    a = adj_ref[...].astype(jnp.bfloat16)
    abf_ref[...] = a
    h = jnp.maximum(
        jnp.dot(a, u1_s[...], preferred_element_type=jnp.float32)
        + b1_ref[...], 0.0)
    u2_ref[...] = jnp.dot(h, w2_ref[...], preferred_element_type=jnp.float32
                          ).astype(jnp.bfloat16)


def _layer23_kernel(abf_ref, atail_ref, u2_ref, b2_ref, w3_ref, b3_ref,
                    wr_ref, br_ref, o_ref, s_ref):
    l = pl.program_id(0)
    i = pl.program_id(1)
    row = pl.ds(i * BM2, BM2)

    @pl.when(l == 0)
    def _layer2():
        h = jnp.maximum(
            jnp.dot(abf_ref[...], u2_ref[...],
                    preferred_element_type=jnp.float32)
            + b2_ref[...], 0.0)
        s_ref[row, 0:1] = jnp.dot(h, w3_ref[...],
                                  preferred_element_type=jnp.float32)

    @pl.when(jnp.logical_and(l == 0, i == NIH - 1))
    def _layer2_tail():
        for c in range(TAIL // 512):
            h = jnp.maximum(
                jnp.dot(atail_ref[pl.ds(c * 512, 512), :], u2_ref[...],
                        preferred_element_type=jnp.float32)
                + b2_ref[...], 0.0)
            s_ref[pl.ds(NH + c * 512, 512), 0:1] = jnp.dot(
                h, w3_ref[...], preferred_element_type=jnp.float32)

    @pl.when(l == 1)
    def _layer3():
        v = s_ref[:, 0:1].astype(jnp.bfloat16)
        h = jnp.maximum(
            jnp.dot(abf_ref[...], v, preferred_element_type=jnp.float32)
            + b3_ref[...], 0.0)
        s_ref[row, 1:2] = h

    @pl.when(jnp.logical_and(l == 1, i == NIH - 1))
    def _layer3_tail_and_readout():
        v = s_ref[:, 0:1].astype(jnp.bfloat16)
        for c in range(TAIL // 512):
            h = jnp.maximum(
                jnp.dot(atail_ref[pl.ds(c * 512, 512), :], v,
                        preferred_element_type=jnp.float32)
                + b3_ref[...], 0.0)
            s_ref[pl.ds(NH + c * 512, 512), 1:2] = h
        o_ref[...] = br_ref[...] + jnp.sum(
            wr_ref[...] * s_ref[:, 1:2].reshape(1, N), keepdims=True)


def kernel(x, adj, W1, b1, W2, b2, W3, b3, Wr, br):
    adj_bf, u2 = pl.pallas_call(
        _layer1_kernel,
        grid=(NI1,),
        in_specs=[
            pl.BlockSpec((BM1, N), lambda i: (i, 0)),
            pl.BlockSpec((N, 128), lambda i: (0, 0)),
            pl.BlockSpec((128, 32), lambda i: (0, 0)),
            pl.BlockSpec((1, 32), lambda i: (0, 0)),
            pl.BlockSpec((32, 16), lambda i: (0, 0)),
        ],
        scratch_shapes=[pltpu.VMEM((N, 32), jnp.bfloat16)],
        out_specs=[
            pl.BlockSpec((BM1, N), lambda i: (i, 0)),
            pl.BlockSpec((BM1, 16), lambda i: (i, 0)),
        ],
        out_shape=[
            jax.ShapeDtypeStruct((N, N), jnp.bfloat16),
            jax.ShapeDtypeStruct((N, 16), jnp.bfloat16),
        ],
    )(adj, x, W1, b1.reshape(1, 32), W2)
    out = pl.pallas_call(
        _layer23_kernel,
        grid=(2, NIH),
        in_specs=[
            pl.BlockSpec((BM2, N), lambda l, i: (i, 0)),
            pl.BlockSpec((TAIL, N), lambda l, i: (NH // TAIL, 0)),
            pl.BlockSpec((N, 16), lambda l, i: (0, 0)),
            pl.BlockSpec((1, 16), lambda l, i: (0, 0)),
            pl.BlockSpec((16, 1), lambda l, i: (0, 0)),
            pl.BlockSpec((1, 1), lambda l, i: (0, 0)),
            pl.BlockSpec((1, N), lambda l, i: (0, 0)),
            pl.BlockSpec((1, 1), lambda l, i: (0, 0)),
        ],
        out_specs=pl.BlockSpec((1, 1), lambda l, i: (0, 0)),
        out_shape=jax.ShapeDtypeStruct((1, 1), jnp.float32),
        scratch_shapes=[
            pltpu.VMEM((N, 128), jnp.float32),
        ],
    )(adj_bf, adj_bf, u2, b2.reshape(1, 16), W3, b3.reshape(1, 1), Wr,
      br.reshape(1, 1))
    return out.reshape(1)
